# baseline (speedup 1.0000x reference)
import jax
import jax.numpy as jnp
from jax.experimental import pallas as pl


_BN = 256
D = 2048


def _add_body(x_ref, pos_ref, out_ref):
    out_ref[...] = x_ref[...] + pos_ref[...][None, :, :]


def kernel(x, pos_embed):
    B, N, D_ = x.shape
    nj = N // _BN
    return pl.pallas_call(
        _add_body,
        grid=(nj,),
        in_specs=[
            pl.BlockSpec((B, _BN, D), lambda j: (0, j, 0)),
            pl.BlockSpec((_BN, D), lambda j: (j, 0)),
        ],
        out_specs=pl.BlockSpec((B, _BN, D), lambda j: (0, j, 0)),
        out_shape=jax.ShapeDtypeStruct((B, N, D), x.dtype),
    )(x, pos_embed)


# speedup vs baseline: 1.0071x; 1.0071x over previous
import jax
import jax.numpy as jnp
from jax.experimental import pallas as pl


_BN = 1024
D = 2048


def _add_body(x_ref, pos_ref, out_ref):
    out_ref[...] = x_ref[...] + pos_ref[...][None, :, :]


def kernel(x, pos_embed):
    B, N, D_ = x.shape
    nj = N // _BN
    return pl.pallas_call(
        _add_body,
        grid=(nj, B),
        in_specs=[
            pl.BlockSpec((1, _BN, D), lambda j, b: (b, j, 0)),
            pl.BlockSpec((_BN, D), lambda j, b: (j, 0)),
        ],
        out_specs=pl.BlockSpec((1, _BN, D), lambda j, b: (b, j, 0)),
        out_shape=jax.ShapeDtypeStruct((B, N, D), x.dtype),
    )(x, pos_embed)
